# Initial kernel scaffold; baseline (speedup 1.0000x reference)
import jax
import jax.numpy as jnp
from jax.experimental import pallas as pl


def kernel(x, edge_index, params):
    raise NotImplementedError("write your pallas kernel here")


# speedup vs baseline: 5.2262x; 5.2262x over previous
import functools

import jax
import jax.numpy as jnp
import numpy as np
from jax.experimental import pallas as pl

N = 10000
E = 320000
D = 128
H = 8
C = D // H
L = 3

_BM = 1000


def _linear_body(h_ref, w_ref, b_ref, o_ref):
    o_ref[...] = (
        jnp.dot(h_ref[...], w_ref[...], preferred_element_type=jnp.float32)
        + b_ref[...]
    )


def _linear(h, W, b):
    m, kdim = h.shape
    n = W.shape[1]
    return pl.pallas_call(
        _linear_body,
        grid=(m // _BM,),
        in_specs=[
            pl.BlockSpec((_BM, kdim), lambda i: (i, 0)),
            pl.BlockSpec((kdim, n), lambda i: (0, 0)),
            pl.BlockSpec((1, n), lambda i: (0, 0)),
        ],
        out_specs=pl.BlockSpec((_BM, n), lambda i: (i, 0)),
        out_shape=jax.ShapeDtypeStruct((m, n), jnp.float32),
    )(h, W, b.reshape(1, n))


def _conv_layer(h, src, dst, p, l):
    q = _linear(h, p['Wq%d' % l], p['bq%d' % l]).reshape(N, H, C)
    k = _linear(h, p['Wk%d' % l], p['bk%d' % l]).reshape(N, H, C)
    v = _linear(h, p['Wv%d' % l], p['bv%d' % l]).reshape(N, H, C)
    s = _linear(h, p['Ws%d' % l], p['bs%d' % l])

    logits = (q[dst] * k[src]).sum(-1) * (1.0 / np.sqrt(C))
    shift = jnp.maximum(jnp.max(logits, axis=0) - 50.0, 0.0)
    ex = jnp.exp(logits - shift)
    denom = jax.ops.segment_sum(ex, dst, num_segments=N)
    w = ex[:, :, None] * v[src]
    acc = jax.ops.segment_sum(w.reshape(E, D), dst, num_segments=N)
    out = acc.reshape(N, H, C) / (denom + 1e-16)[:, :, None]
    return out.reshape(N, D) + s


def kernel(x, edge_index, params):
    src, dst = edge_index[0], edge_index[1]
    h = _linear(x, params['W_in'], params['b_in'])
    for l in range(L):
        h = jax.nn.relu(_conv_layer(h, src, dst, params, l))
    return _linear(h, params['W_out'], params['b_out'])


# speedup vs baseline: 30.4880x; 5.8337x over previous
import functools

import jax
import jax.numpy as jnp
import numpy as np
from jax import lax
from jax.experimental import pallas as pl
from jax.experimental.pallas import tpu as pltpu
from jax.experimental.pallas import tpu_sc as plsc

N = 10000
E = 320000
D = 128
H = 8
C = D // H
L = 3

NC = 2
NS = 16
NW = NC * NS
EW = E // NW
CH = 80
NJ = EW // CH
NP = 10240
NPS = NP // NS

_BM = 1000
_BE = 4000

_mesh = plsc.VectorSubcoreMesh(core_axis_name="c", subcore_axis_name="s")


def _sc_gather3(q2, k2, v2, dst3, src3):
    row_t = jax.ShapeDtypeStruct((E, D), jnp.float32)

    @functools.partial(
        pl.kernel,
        out_type=[row_t, row_t, row_t],
        mesh=_mesh,
        scratch_types=[
            pltpu.VMEM((NJ, CH), jnp.int32),
            pltpu.VMEM((NJ, CH), jnp.int32),
            pltpu.VMEM((CH, D), jnp.float32),
            pltpu.VMEM((CH, D), jnp.float32),
            pltpu.VMEM((CH, D), jnp.float32),
        ],
    )
    def body(q_hbm, k_hbm, v_hbm, dst_hbm, src_hbm, qd_hbm, ks_hbm, vs_hbm,
             idxd, idxs, rq, rk, rv):
        cid = lax.axis_index("c")
        sid = lax.axis_index("s")
        wid = cid * NS + sid
        base = wid * EW
        pltpu.sync_copy(dst_hbm.at[wid], idxd)
        pltpu.sync_copy(src_hbm.at[wid], idxs)

        @pl.loop(0, NJ)
        def _(j):
            o = base + j * CH
            pltpu.sync_copy(q_hbm.at[idxd.at[j]], rq)
            pltpu.sync_copy(rq, qd_hbm.at[pl.ds(o, CH)])
            pltpu.sync_copy(k_hbm.at[idxs.at[j]], rk)
            pltpu.sync_copy(rk, ks_hbm.at[pl.ds(o, CH)])
            pltpu.sync_copy(v_hbm.at[idxs.at[j]], rv)
            pltpu.sync_copy(rv, vs_hbm.at[pl.ds(o, CH)])

    return body(q2, k2, v2, dst3, src3)


def _sc_scatter(ex16, w, dst3, z16, z128):
    @functools.partial(
        pl.kernel,
        out_type=[
            jax.ShapeDtypeStruct((NC, NP, 16), jnp.float32),
            jax.ShapeDtypeStruct((NC, NP, D), jnp.float32),
        ],
        mesh=_mesh,
        scratch_types=[
            pltpu.VMEM_SHARED((NP, 16), jnp.float32),
            pltpu.VMEM_SHARED((NP, D), jnp.float32),
            pltpu.VMEM((NJ, CH), jnp.int32),
            pltpu.VMEM((CH, 16), jnp.float32),
            pltpu.VMEM((CH, D), jnp.float32),
        ],
    )
    def body(ex_hbm, w_hbm, dst_hbm, z16_hbm, z128_hbm, den_hbm, out_hbm,
             accd, acco, idx, rex, rw):
        cid = lax.axis_index("c")
        sid = lax.axis_index("s")
        wid = cid * NS + sid
        base = wid * EW
        stripe = pl.ds(sid * NPS, NPS)
        pltpu.sync_copy(z16_hbm, accd.at[stripe])
        pltpu.sync_copy(z128_hbm, acco.at[stripe])
        pltpu.sync_copy(dst_hbm.at[wid], idx)
        plsc.subcore_barrier()

        @pl.loop(0, NJ)
        def _(j):
            o = base + j * CH
            pltpu.sync_copy(ex_hbm.at[pl.ds(o, CH)], rex)
            pltpu.sync_copy(rex, accd.at[idx.at[j]], add=True)
            pltpu.sync_copy(w_hbm.at[pl.ds(o, CH)], rw)
            pltpu.sync_copy(rw, acco.at[idx.at[j]], add=True)

        plsc.subcore_barrier()
        pltpu.sync_copy(accd.at[stripe], den_hbm.at[cid].at[stripe])
        pltpu.sync_copy(acco.at[stripe], out_hbm.at[cid].at[stripe])

    return body(ex16, w, dst3, z16, z128)


def _linear_body(h_ref, w_ref, b_ref, o_ref):
    o_ref[...] = (
        jnp.dot(h_ref[...], w_ref[...], preferred_element_type=jnp.float32)
        + b_ref[...]
    )


def _linear(h, W, b):
    m, kdim = h.shape
    n = W.shape[1]
    return pl.pallas_call(
        _linear_body,
        grid=(m // _BM,),
        in_specs=[
            pl.BlockSpec((_BM, kdim), lambda i: (i, 0)),
            pl.BlockSpec((kdim, n), lambda i: (0, 0)),
            pl.BlockSpec((1, n), lambda i: (0, 0)),
        ],
        out_specs=pl.BlockSpec((_BM, n), lambda i: (i, 0)),
        out_shape=jax.ShapeDtypeStruct((m, n), jnp.float32),
    )(h, W, b.reshape(1, n))


def _proj4_body(h_ref, w_ref, b_ref, q_ref, k_ref, v_ref, s_ref):
    h = h_ref[...]
    for t, o_ref in enumerate((q_ref, k_ref, v_ref, s_ref)):
        o_ref[...] = (
            jnp.dot(h, w_ref[t], preferred_element_type=jnp.float32)
            + b_ref[t]
        )


def _proj4(h, Ws, bs):
    ot = jax.ShapeDtypeStruct((N, D), jnp.float32)
    return pl.pallas_call(
        _proj4_body,
        grid=(N // _BM,),
        in_specs=[
            pl.BlockSpec((_BM, D), lambda i: (i, 0)),
            pl.BlockSpec((4, D, D), lambda i: (0, 0, 0)),
            pl.BlockSpec((4, 1, D), lambda i: (0, 0, 0)),
        ],
        out_specs=[pl.BlockSpec((_BM, D), lambda i: (i, 0))] * 4,
        out_shape=[ot, ot, ot, ot],
    )(h, Ws, bs)


def _logits_body(qd_ref, ks_ref, sel_ref, lg_ref, mx_ref):
    i = pl.program_id(0)
    prod = qd_ref[...] * ks_ref[...]
    lg = jnp.dot(prod, sel_ref[...], preferred_element_type=jnp.float32)
    lg_ref[...] = lg
    bmx = jnp.max(lg, axis=0, keepdims=True)

    @pl.when(i == 0)
    def _():
        mx_ref[...] = bmx

    @pl.when(i > 0)
    def _():
        mx_ref[...] = jnp.maximum(mx_ref[...], bmx)


def _logits(qd, ks, sel):
    return pl.pallas_call(
        _logits_body,
        grid=(E // _BE,),
        in_specs=[
            pl.BlockSpec((_BE, D), lambda i: (i, 0)),
            pl.BlockSpec((_BE, D), lambda i: (i, 0)),
            pl.BlockSpec((D, H), lambda i: (0, 0)),
        ],
        out_specs=[
            pl.BlockSpec((_BE, H), lambda i: (i, 0)),
            pl.BlockSpec((1, H), lambda i: (0, 0)),
        ],
        out_shape=[
            jax.ShapeDtypeStruct((E, H), jnp.float32),
            jax.ShapeDtypeStruct((1, H), jnp.float32),
        ],
    )(qd, ks, sel)


def _weight_body(lg_ref, sh_ref, vs_ref, exp_ref, ex_ref, w_ref):
    ex = jnp.exp(lg_ref[...] - sh_ref[...])
    ex_ref[:, :H] = ex
    ex_ref[:, H:] = jnp.zeros_like(ex)
    w_ref[...] = vs_ref[...] * jnp.dot(
        ex, exp_ref[...], preferred_element_type=jnp.float32
    )


def _weight(lg, shift, vs, expand):
    return pl.pallas_call(
        _weight_body,
        grid=(E // _BE,),
        in_specs=[
            pl.BlockSpec((_BE, H), lambda i: (i, 0)),
            pl.BlockSpec((1, H), lambda i: (0, 0)),
            pl.BlockSpec((_BE, D), lambda i: (i, 0)),
            pl.BlockSpec((H, D), lambda i: (0, 0)),
        ],
        out_specs=[
            pl.BlockSpec((_BE, 16), lambda i: (i, 0)),
            pl.BlockSpec((_BE, D), lambda i: (i, 0)),
        ],
        out_shape=[
            jax.ShapeDtypeStruct((E, 16), jnp.float32),
            jax.ShapeDtypeStruct((E, D), jnp.float32),
        ],
    )(lg, shift, vs, expand)


def _finish_body(p0_ref, p1_ref, d0_ref, d1_ref, s_ref, exp_ref, o_ref):
    den = (d0_ref[...] + d1_ref[...])[:, :H] + 1e-16
    r = jnp.dot(1.0 / den, exp_ref[...], preferred_element_type=jnp.float32)
    o_ref[...] = jax.nn.relu((p0_ref[...] + p1_ref[...]) * r + s_ref[...])


def _finish(p0, p1, d0, d1, s, expand):
    return pl.pallas_call(
        _finish_body,
        grid=(N // _BM,),
        in_specs=[
            pl.BlockSpec((_BM, D), lambda i: (i, 0)),
            pl.BlockSpec((_BM, D), lambda i: (i, 0)),
            pl.BlockSpec((_BM, 16), lambda i: (i, 0)),
            pl.BlockSpec((_BM, 16), lambda i: (i, 0)),
            pl.BlockSpec((_BM, D), lambda i: (i, 0)),
            pl.BlockSpec((H, D), lambda i: (0, 0)),
        ],
        out_specs=pl.BlockSpec((_BM, D), lambda i: (i, 0)),
        out_shape=jax.ShapeDtypeStruct((N, D), jnp.float32),
    )(p0, p1, d0, d1, s, expand)


def kernel(x, edge_index, params):
    dst3 = edge_index[1].reshape(NW, NJ, CH)
    src3 = edge_index[0].reshape(NW, NJ, CH)
    z16 = jnp.zeros((NPS, 16), jnp.float32)
    z128 = jnp.zeros((NPS, D), jnp.float32)
    heads = np.arange(D) // C
    sel = jnp.asarray(
        (heads[:, None] == np.arange(H)[None, :]) / np.sqrt(C), jnp.float32
    )
    expand = jnp.asarray(
        (np.arange(H)[:, None] == heads[None, :]).astype(np.float32)
    )

    h = _linear(x, params['W_in'], params['b_in'])
    for l in range(L):
        Wst = jnp.stack([params['W%s%d' % (nm, l)] for nm in 'qkvs'])
        bst = jnp.stack(
            [params['b%s%d' % (nm, l)].reshape(1, D) for nm in 'qkvs']
        )
        q, k, v, s = _proj4(h, Wst, bst)
        qd, ks, vs = _sc_gather3(q, k, v, dst3, src3)
        lg, mx = _logits(qd, ks, sel)
        shift = jnp.maximum(mx - 50.0, 0.0)
        ex16, w = _weight(lg, shift, vs, expand)
        den_p, out_p = _sc_scatter(ex16, w, dst3, z16, z128)
        h = _finish(out_p[0, :N], out_p[1, :N], den_p[0, :N], den_p[1, :N],
                    s, expand)
    return _linear(h, params['W_out'], params['b_out'])


# speedup vs baseline: 41.8844x; 1.3738x over previous
import functools

import jax
import jax.numpy as jnp
import numpy as np
from jax import lax
from jax.experimental import pallas as pl
from jax.experimental.pallas import tpu as pltpu
from jax.experimental.pallas import tpu_sc as plsc

N = 10000
E = 320000
D = 128
H = 8
C = D // H
L = 3

NC = 2
NS = 16
NW = NC * NS
EW = E // NW
CH = 40
NJ = EW // CH
NP = 10240
NPS = NP // NS

_BM = 1000
_BE = 4000

_mesh = plsc.VectorSubcoreMesh(core_axis_name="c", subcore_axis_name="s")


def _sc_gather(q2, kv2, dst3, src3):
    @functools.partial(
        pl.kernel,
        out_type=[
            jax.ShapeDtypeStruct((E, D), jnp.float32),
            jax.ShapeDtypeStruct((E, 2 * D), jnp.float32),
        ],
        mesh=_mesh,
        scratch_types=[
            pltpu.VMEM((NJ, CH), jnp.int32),
            pltpu.VMEM((NJ, CH), jnp.int32),
            pltpu.VMEM((CH, D), jnp.float32),
            pltpu.VMEM((CH, D), jnp.float32),
            pltpu.VMEM((CH, 2 * D), jnp.float32),
            pltpu.VMEM((CH, 2 * D), jnp.float32),
            pltpu.SemaphoreType.DMA((2,)),
            pltpu.SemaphoreType.DMA((2,)),
        ],
    )
    def body(q_hbm, kv_hbm, dst_hbm, src_hbm, qd_hbm, kvs_hbm,
             idxd, idxs, bq0, bq1, bkv0, bkv1, gsem, wsem):
        cid = lax.axis_index("c")
        sid = lax.axis_index("s")
        wid = cid * NS + sid
        base = wid * EW
        pltpu.sync_copy(dst_hbm.at[wid], idxd)
        pltpu.sync_copy(src_hbm.at[wid], idxs)
        bq = (bq0, bq1)
        bkv = (bkv0, bkv1)

        def gs(b, j):
            pltpu.async_copy(q_hbm.at[idxd.at[j]], bq[b], gsem.at[b])
            pltpu.async_copy(kv_hbm.at[idxs.at[j]], bkv[b], gsem.at[b])

        def gw(b, j):
            pltpu.make_async_copy(q_hbm.at[idxd.at[j]], bq[b],
                                  gsem.at[b]).wait()
            pltpu.make_async_copy(kv_hbm.at[idxs.at[j]], bkv[b],
                                  gsem.at[b]).wait()

        def ws(b, j):
            o = base + j * CH
            pltpu.async_copy(bq[b], qd_hbm.at[pl.ds(o, CH)], wsem.at[b])
            pltpu.async_copy(bkv[b], kvs_hbm.at[pl.ds(o, CH)], wsem.at[b])

        def ww(b, j):
            o = base + j * CH
            pltpu.make_async_copy(bq[b], qd_hbm.at[pl.ds(o, CH)],
                                  wsem.at[b]).wait()
            pltpu.make_async_copy(bkv[b], kvs_hbm.at[pl.ds(o, CH)],
                                  wsem.at[b]).wait()

        gs(0, 0)
        gs(1, 1)
        gw(0, 0)
        ws(0, 0)

        @pl.loop(0, (NJ - 2) // 2)
        def _(jj):
            p = 2 + 2 * jj
            ww(0, p - 2)
            gs(0, p)
            gw(1, p - 1)
            ws(1, p - 1)
            ww(1, p - 1)
            gs(1, p + 1)
            gw(0, p)
            ws(0, p)

        ww(0, NJ - 2)
        gw(1, NJ - 1)
        ws(1, NJ - 1)
        ww(1, NJ - 1)

    return body(q2, kv2, dst3, src3)


def _sc_scatter(ex16, w, dst3, z16, z128):
    @functools.partial(
        pl.kernel,
        out_type=[
            jax.ShapeDtypeStruct((NC, NP, 16), jnp.float32),
            jax.ShapeDtypeStruct((NC, NP, D), jnp.float32),
        ],
        mesh=_mesh,
        scratch_types=[
            pltpu.VMEM_SHARED((NP, 16), jnp.float32),
            pltpu.VMEM_SHARED((NP, D), jnp.float32),
            pltpu.VMEM((1, CH), jnp.int32),
            pltpu.VMEM((1, CH), jnp.int32),
            pltpu.VMEM((CH, 16), jnp.float32),
            pltpu.VMEM((CH, 16), jnp.float32),
            pltpu.VMEM((CH, D), jnp.float32),
            pltpu.VMEM((CH, D), jnp.float32),
            pltpu.SemaphoreType.DMA((2,)),
            pltpu.SemaphoreType.DMA((2,)),
        ],
    )
    def body(ex_hbm, w_hbm, dst_hbm, z16_hbm, z128_hbm, den_hbm, out_hbm,
             accd, acco, bi0, bi1, be0, be1, bw0, bw1, lsem, ssem):
        cid = lax.axis_index("c")
        sid = lax.axis_index("s")
        wid = cid * NS + sid
        base = wid * EW
        stripe = pl.ds(sid * NPS, NPS)
        pltpu.sync_copy(z16_hbm, accd.at[stripe])
        pltpu.sync_copy(z128_hbm, acco.at[stripe])
        plsc.subcore_barrier()
        bi = (bi0, bi1)
        be = (be0, be1)
        bw = (bw0, bw1)

        def ls(b, j):
            o = base + j * CH
            pltpu.async_copy(dst_hbm.at[wid].at[pl.ds(j, 1)], bi[b],
                             lsem.at[b])
            pltpu.async_copy(ex_hbm.at[pl.ds(o, CH)], be[b], lsem.at[b])
            pltpu.async_copy(w_hbm.at[pl.ds(o, CH)], bw[b], lsem.at[b])

        def lw(b, j):
            o = base + j * CH
            pltpu.make_async_copy(dst_hbm.at[wid].at[pl.ds(j, 1)], bi[b],
                                  lsem.at[b]).wait()
            pltpu.make_async_copy(ex_hbm.at[pl.ds(o, CH)], be[b],
                                  lsem.at[b]).wait()
            pltpu.make_async_copy(w_hbm.at[pl.ds(o, CH)], bw[b],
                                  lsem.at[b]).wait()

        def scs(b, j):
            pltpu.async_copy(be[b], accd.at[bi[b].at[0]], ssem.at[b],
                             add=True)
            pltpu.async_copy(bw[b], acco.at[bi[b].at[0]], ssem.at[b],
                             add=True)

        def scw(b, j):
            pltpu.make_async_copy(be[b], accd.at[bi[b].at[0]],
                                  ssem.at[b]).wait()
            pltpu.make_async_copy(bw[b], acco.at[bi[b].at[0]],
                                  ssem.at[b]).wait()

        ls(0, 0)
        ls(1, 1)

        @pl.loop(0, NJ // 2 - 1)
        def _(jj):
            p = 2 * jj
            lw(0, p)
            scs(0, p)
            lw(1, p + 1)
            scs(1, p + 1)
            scw(0, p)
            ls(0, p + 2)
            scw(1, p + 1)
            ls(1, p + 3)

        lw(0, NJ - 2)
        scs(0, NJ - 2)
        lw(1, NJ - 1)
        scs(1, NJ - 1)
        scw(0, NJ - 2)
        scw(1, NJ - 1)

        plsc.subcore_barrier()
        pltpu.sync_copy(accd.at[stripe], den_hbm.at[cid].at[stripe])
        pltpu.sync_copy(acco.at[stripe], out_hbm.at[cid].at[stripe])

    return body(ex16, w, dst3, z16, z128)


def _linear_body(h_ref, w_ref, b_ref, o_ref):
    o_ref[...] = (
        jnp.dot(h_ref[...], w_ref[...], preferred_element_type=jnp.float32)
        + b_ref[...]
    )


def _linear(h, W, b):
    m, kdim = h.shape
    n = W.shape[1]
    return pl.pallas_call(
        _linear_body,
        grid=(m // _BM,),
        in_specs=[
            pl.BlockSpec((_BM, kdim), lambda i: (i, 0)),
            pl.BlockSpec((kdim, n), lambda i: (0, 0)),
            pl.BlockSpec((1, n), lambda i: (0, 0)),
        ],
        out_specs=pl.BlockSpec((_BM, n), lambda i: (i, 0)),
        out_shape=jax.ShapeDtypeStruct((m, n), jnp.float32),
    )(h, W, b.reshape(1, n))


def _proj3_body(h_ref, wq_ref, bq_ref, wkv_ref, bkv_ref, ws_ref, bs_ref,
                q_ref, kv_ref, s_ref):
    h = h_ref[...]
    q_ref[...] = (
        jnp.dot(h, wq_ref[...], preferred_element_type=jnp.float32)
        + bq_ref[...]
    )
    kv_ref[...] = (
        jnp.dot(h, wkv_ref[...], preferred_element_type=jnp.float32)
        + bkv_ref[...]
    )
    s_ref[...] = (
        jnp.dot(h, ws_ref[...], preferred_element_type=jnp.float32)
        + bs_ref[...]
    )


def _proj3(h, Wq, bq, Wkv, bkv, Ws, bs):
    return pl.pallas_call(
        _proj3_body,
        grid=(N // _BM,),
        in_specs=[
            pl.BlockSpec((_BM, D), lambda i: (i, 0)),
            pl.BlockSpec((D, D), lambda i: (0, 0)),
            pl.BlockSpec((1, D), lambda i: (0, 0)),
            pl.BlockSpec((D, 2 * D), lambda i: (0, 0)),
            pl.BlockSpec((1, 2 * D), lambda i: (0, 0)),
            pl.BlockSpec((D, D), lambda i: (0, 0)),
            pl.BlockSpec((1, D), lambda i: (0, 0)),
        ],
        out_specs=[
            pl.BlockSpec((_BM, D), lambda i: (i, 0)),
            pl.BlockSpec((_BM, 2 * D), lambda i: (i, 0)),
            pl.BlockSpec((_BM, D), lambda i: (i, 0)),
        ],
        out_shape=[
            jax.ShapeDtypeStruct((N, D), jnp.float32),
            jax.ShapeDtypeStruct((N, 2 * D), jnp.float32),
            jax.ShapeDtypeStruct((N, D), jnp.float32),
        ],
    )(h, Wq, bq.reshape(1, D), Wkv, bkv.reshape(1, 2 * D),
      Ws, bs.reshape(1, D))


def _logits_body(qd_ref, ks_ref, sel_ref, lg_ref, mx_ref):
    i = pl.program_id(0)
    prod = qd_ref[...] * ks_ref[...]
    lg = jnp.dot(prod, sel_ref[...], preferred_element_type=jnp.float32)
    lg_ref[...] = lg
    bmx = jnp.max(lg, axis=0, keepdims=True)

    @pl.when(i == 0)
    def _():
        mx_ref[...] = bmx

    @pl.when(i > 0)
    def _():
        mx_ref[...] = jnp.maximum(mx_ref[...], bmx)


def _logits(qd, kvs, sel):
    return pl.pallas_call(
        _logits_body,
        grid=(E // _BE,),
        in_specs=[
            pl.BlockSpec((_BE, D), lambda i: (i, 0)),
            pl.BlockSpec((_BE, D), lambda i: (i, 0)),
            pl.BlockSpec((D, H), lambda i: (0, 0)),
        ],
        out_specs=[
            pl.BlockSpec((_BE, H), lambda i: (i, 0)),
            pl.BlockSpec((1, H), lambda i: (0, 0)),
        ],
        out_shape=[
            jax.ShapeDtypeStruct((E, H), jnp.float32),
            jax.ShapeDtypeStruct((1, H), jnp.float32),
        ],
    )(qd, kvs, sel)


def _weight_body(lg_ref, sh_ref, vs_ref, exp_ref, ex_ref, w_ref):
    ex = jnp.exp(lg_ref[...] - sh_ref[...])
    ex_ref[:, :H] = ex
    ex_ref[:, H:] = jnp.zeros_like(ex)
    w_ref[...] = vs_ref[...] * jnp.dot(
        ex, exp_ref[...], preferred_element_type=jnp.float32
    )


def _weight(lg, shift, kvs, expand):
    return pl.pallas_call(
        _weight_body,
        grid=(E // _BE,),
        in_specs=[
            pl.BlockSpec((_BE, H), lambda i: (i, 0)),
            pl.BlockSpec((1, H), lambda i: (0, 0)),
            pl.BlockSpec((_BE, D), lambda i: (i, 1)),
            pl.BlockSpec((H, D), lambda i: (0, 0)),
        ],
        out_specs=[
            pl.BlockSpec((_BE, 16), lambda i: (i, 0)),
            pl.BlockSpec((_BE, D), lambda i: (i, 0)),
        ],
        out_shape=[
            jax.ShapeDtypeStruct((E, 16), jnp.float32),
            jax.ShapeDtypeStruct((E, D), jnp.float32),
        ],
    )(lg, shift, kvs, expand)


def _finish_body(p0_ref, p1_ref, d0_ref, d1_ref, s_ref, exp_ref, o_ref):
    den = (d0_ref[...] + d1_ref[...])[:, :H] + 1e-16
    r = jnp.dot(1.0 / den, exp_ref[...], preferred_element_type=jnp.float32)
    o_ref[...] = jax.nn.relu((p0_ref[...] + p1_ref[...]) * r + s_ref[...])


def _finish(p0, p1, d0, d1, s, expand):
    return pl.pallas_call(
        _finish_body,
        grid=(N // _BM,),
        in_specs=[
            pl.BlockSpec((_BM, D), lambda i: (i, 0)),
            pl.BlockSpec((_BM, D), lambda i: (i, 0)),
            pl.BlockSpec((_BM, 16), lambda i: (i, 0)),
            pl.BlockSpec((_BM, 16), lambda i: (i, 0)),
            pl.BlockSpec((_BM, D), lambda i: (i, 0)),
            pl.BlockSpec((H, D), lambda i: (0, 0)),
        ],
        out_specs=pl.BlockSpec((_BM, D), lambda i: (i, 0)),
        out_shape=jax.ShapeDtypeStruct((N, D), jnp.float32),
    )(p0, p1, d0, d1, s, expand)


def kernel(x, edge_index, params):
    dst3 = edge_index[1].reshape(NW, NJ, CH)
    src3 = edge_index[0].reshape(NW, NJ, CH)
    z16 = jnp.zeros((NPS, 16), jnp.float32)
    z128 = jnp.zeros((NPS, D), jnp.float32)
    heads = np.arange(D) // C
    sel = jnp.asarray(
        (heads[:, None] == np.arange(H)[None, :]) / np.sqrt(C), jnp.float32
    )
    expand = jnp.asarray(
        (np.arange(H)[:, None] == heads[None, :]).astype(np.float32)
    )

    h = _linear(x, params['W_in'], params['b_in'])
    for l in range(L):
        Wkv = jnp.concatenate(
            [params['Wk%d' % l], params['Wv%d' % l]], axis=1
        )
        bkv = jnp.concatenate([params['bk%d' % l], params['bv%d' % l]])
        q, kv, s = _proj3(h, params['Wq%d' % l], params['bq%d' % l],
                          Wkv, bkv, params['Ws%d' % l], params['bs%d' % l])
        qd, kvs = _sc_gather(q, kv, dst3, src3)
        lg, mx = _logits(qd, kvs, sel)
        shift = jnp.maximum(mx - 50.0, 0.0)
        ex16, w = _weight(lg, shift, kvs, expand)
        den_p, out_p = _sc_scatter(ex16, w, dst3, z16, z128)
        h = _finish(out_p[0, :N], out_p[1, :N], den_p[0, :N], den_p[1, :N],
                    s, expand)
    return _linear(h, params['W_out'], params['b_out'])


# speedup vs baseline: 47.8736x; 1.1430x over previous
import functools

import jax
import jax.numpy as jnp
import numpy as np
from jax import lax
from jax.experimental import pallas as pl
from jax.experimental.pallas import tpu as pltpu
from jax.experimental.pallas import tpu_sc as plsc

N = 10000
E = 320000
D = 128
H = 8
C = D // H
L = 3

NC = 2
NS = 16
NW = NC * NS
NHALF = 2
E2 = E // NHALF
EW = E2 // NW
CH = 40
NJ = EW // CH
NP = 10240
NPS = NP // NS

_BM = 1000
_BE = 4000

_mesh = plsc.VectorSubcoreMesh(core_axis_name="c", subcore_axis_name="s")

assert NJ % 2 == 1


def _sc_gather(q2, kv2, dst3, src3):
    @functools.partial(
        pl.kernel,
        out_type=[
            jax.ShapeDtypeStruct((E2, D), jnp.float32),
            jax.ShapeDtypeStruct((E2, 2 * D), jnp.float32),
        ],
        mesh=_mesh,
        scratch_types=[
            pltpu.VMEM((NJ, CH), jnp.int32),
            pltpu.VMEM((NJ, CH), jnp.int32),
            pltpu.VMEM((CH, D), jnp.float32),
            pltpu.VMEM((CH, D), jnp.float32),
            pltpu.VMEM((CH, 2 * D), jnp.float32),
            pltpu.VMEM((CH, 2 * D), jnp.float32),
            pltpu.SemaphoreType.DMA((2,)),
            pltpu.SemaphoreType.DMA((2,)),
        ],
    )
    def body(q_hbm, kv_hbm, dst_hbm, src_hbm, qd_hbm, kvs_hbm,
             idxd, idxs, bq0, bq1, bkv0, bkv1, gsem, wsem):
        cid = lax.axis_index("c")
        sid = lax.axis_index("s")
        wid = cid * NS + sid
        base = wid * EW
        pltpu.sync_copy(dst_hbm.at[wid], idxd)
        pltpu.sync_copy(src_hbm.at[wid], idxs)
        bq = (bq0, bq1)
        bkv = (bkv0, bkv1)

        def gs(b, j):
            pltpu.async_copy(q_hbm.at[idxd.at[j]], bq[b], gsem.at[b])
            pltpu.async_copy(kv_hbm.at[idxs.at[j]], bkv[b], gsem.at[b])

        def gw(b, j):
            pltpu.make_async_copy(q_hbm.at[idxd.at[j]], bq[b],
                                  gsem.at[b]).wait()
            pltpu.make_async_copy(kv_hbm.at[idxs.at[j]], bkv[b],
                                  gsem.at[b]).wait()

        def ws(b, j):
            o = base + j * CH
            pltpu.async_copy(bq[b], qd_hbm.at[pl.ds(o, CH)], wsem.at[b])
            pltpu.async_copy(bkv[b], kvs_hbm.at[pl.ds(o, CH)], wsem.at[b])

        def ww(b, j):
            o = base + j * CH
            pltpu.make_async_copy(bq[b], qd_hbm.at[pl.ds(o, CH)],
                                  wsem.at[b]).wait()
            pltpu.make_async_copy(bkv[b], kvs_hbm.at[pl.ds(o, CH)],
                                  wsem.at[b]).wait()

        gs(0, 0)
        gs(1, 1)
        gw(0, 0)
        ws(0, 0)

        @pl.loop(0, (NJ - 2) // 2)
        def _(jj):
            p = 2 + 2 * jj
            ww(0, p - 2)
            gs(0, p)
            gw(1, p - 1)
            ws(1, p - 1)
            ww(1, p - 1)
            gs(1, p + 1)
            gw(0, p)
            ws(0, p)

        ww(0, NJ - 3)
        gs(0, NJ - 1)
        gw(1, NJ - 2)
        ws(1, NJ - 2)
        ww(1, NJ - 2)
        gw(0, NJ - 1)
        ws(0, NJ - 1)
        ww(0, NJ - 1)

    return body(q2, kv2, dst3, src3)


def _sc_scatter(ex16, w, dst3, z16, z128):
    @functools.partial(
        pl.kernel,
        out_type=[
            jax.ShapeDtypeStruct((NC, NP, 16), jnp.float32),
            jax.ShapeDtypeStruct((NC, NP, D), jnp.float32),
        ],
        mesh=_mesh,
        scratch_types=[
            pltpu.VMEM_SHARED((NP, 16), jnp.float32),
            pltpu.VMEM_SHARED((NP, D), jnp.float32),
            pltpu.VMEM((1, CH), jnp.int32),
            pltpu.VMEM((1, CH), jnp.int32),
            pltpu.VMEM((CH, 16), jnp.float32),
            pltpu.VMEM((CH, 16), jnp.float32),
            pltpu.VMEM((CH, D), jnp.float32),
            pltpu.VMEM((CH, D), jnp.float32),
            pltpu.SemaphoreType.DMA((2,)),
            pltpu.SemaphoreType.DMA((2,)),
        ],
    )
    def body(ex_hbm, w_hbm, dst_hbm, z16_hbm, z128_hbm, den_hbm, out_hbm,
             accd, acco, bi0, bi1, be0, be1, bw0, bw1, lsem, ssem):
        cid = lax.axis_index("c")
        sid = lax.axis_index("s")
        wid = cid * NS + sid
        base = wid * EW
        stripe = pl.ds(sid * NPS, NPS)
        pltpu.sync_copy(z16_hbm, accd.at[stripe])
        pltpu.sync_copy(z128_hbm, acco.at[stripe])
        plsc.subcore_barrier()
        bi = (bi0, bi1)
        be = (be0, be1)
        bw = (bw0, bw1)

        def ls(b, j):
            o = base + j * CH
            pltpu.async_copy(dst_hbm.at[wid].at[pl.ds(j, 1)], bi[b],
                             lsem.at[b])
            pltpu.async_copy(ex_hbm.at[pl.ds(o, CH)], be[b], lsem.at[b])
            pltpu.async_copy(w_hbm.at[pl.ds(o, CH)], bw[b], lsem.at[b])

        def lw(b, j):
            o = base + j * CH
            pltpu.make_async_copy(dst_hbm.at[wid].at[pl.ds(j, 1)], bi[b],
                                  lsem.at[b]).wait()
            pltpu.make_async_copy(ex_hbm.at[pl.ds(o, CH)], be[b],
                                  lsem.at[b]).wait()
            pltpu.make_async_copy(w_hbm.at[pl.ds(o, CH)], bw[b],
                                  lsem.at[b]).wait()

        def scs(b, j):
            pltpu.async_copy(be[b], accd.at[bi[b].at[0]], ssem.at[b],
                             add=True)
            pltpu.async_copy(bw[b], acco.at[bi[b].at[0]], ssem.at[b],
                             add=True)

        def scw(b, j):
            pltpu.make_async_copy(be[b], accd.at[bi[b].at[0]],
                                  ssem.at[b]).wait()
            pltpu.make_async_copy(bw[b], acco.at[bi[b].at[0]],
                                  ssem.at[b]).wait()

        ls(0, 0)
        ls(1, 1)

        @pl.loop(0, (NJ - 3) // 2)
        def _(jj):
            p = 2 * jj
            lw(0, p)
            scs(0, p)
            lw(1, p + 1)
            scs(1, p + 1)
            scw(0, p)
            ls(0, p + 2)
            scw(1, p + 1)
            ls(1, p + 3)

        p = NJ - 3
        lw(0, p)
        scs(0, p)
        lw(1, p + 1)
        scs(1, p + 1)
        scw(0, p)
        ls(0, p + 2)
        scw(1, p + 1)
        lw(0, p + 2)
        scs(0, p + 2)
        scw(0, p + 2)

        plsc.subcore_barrier()
        pltpu.sync_copy(accd.at[stripe], den_hbm.at[cid].at[stripe])
        pltpu.sync_copy(acco.at[stripe], out_hbm.at[cid].at[stripe])

    return body(ex16, w, dst3, z16, z128)


def _linear_body(h_ref, w_ref, b_ref, o_ref):
    o_ref[...] = (
        jnp.dot(h_ref[...], w_ref[...], preferred_element_type=jnp.float32)
        + b_ref[...]
    )


def _linear(h, W, b):
    m, kdim = h.shape
    n = W.shape[1]
    return pl.pallas_call(
        _linear_body,
        grid=(m // _BM,),
        in_specs=[
            pl.BlockSpec((_BM, kdim), lambda i: (i, 0)),
            pl.BlockSpec((kdim, n), lambda i: (0, 0)),
            pl.BlockSpec((1, n), lambda i: (0, 0)),
        ],
        out_specs=pl.BlockSpec((_BM, n), lambda i: (i, 0)),
        out_shape=jax.ShapeDtypeStruct((m, n), jnp.float32),
    )(h, W, b.reshape(1, n))


def _proj3_body(h_ref, wq_ref, bq_ref, wkv_ref, bkv_ref, ws_ref, bs_ref,
                selo_ref, q_ref, kv_ref, s_ref, mq_ref, mk_ref):
    i = pl.program_id(0)
    h = h_ref[...]
    q = jnp.dot(h, wq_ref[...], preferred_element_type=jnp.float32) \
        + bq_ref[...]
    kv = jnp.dot(h, wkv_ref[...], preferred_element_type=jnp.float32) \
        + bkv_ref[...]
    q_ref[...] = q
    kv_ref[...] = kv
    s_ref[...] = (
        jnp.dot(h, ws_ref[...], preferred_element_type=jnp.float32)
        + bs_ref[...]
    )
    k = kv[:, :D]
    mq = jnp.max(jnp.dot(q * q, selo_ref[...],
                         preferred_element_type=jnp.float32),
                 axis=0, keepdims=True)
    mk = jnp.max(jnp.dot(k * k, selo_ref[...],
                         preferred_element_type=jnp.float32),
                 axis=0, keepdims=True)

    @pl.when(i == 0)
    def _():
        mq_ref[...] = mq
        mk_ref[...] = mk

    @pl.when(i > 0)
    def _():
        mq_ref[...] = jnp.maximum(mq_ref[...], mq)
        mk_ref[...] = jnp.maximum(mk_ref[...], mk)


def _proj3(h, Wq, bq, Wkv, bkv, Ws, bs, selo):
    return pl.pallas_call(
        _proj3_body,
        grid=(N // _BM,),
        in_specs=[
            pl.BlockSpec((_BM, D), lambda i: (i, 0)),
            pl.BlockSpec((D, D), lambda i: (0, 0)),
            pl.BlockSpec((1, D), lambda i: (0, 0)),
            pl.BlockSpec((D, 2 * D), lambda i: (0, 0)),
            pl.BlockSpec((1, 2 * D), lambda i: (0, 0)),
            pl.BlockSpec((D, D), lambda i: (0, 0)),
            pl.BlockSpec((1, D), lambda i: (0, 0)),
            pl.BlockSpec((D, H), lambda i: (0, 0)),
        ],
        out_specs=[
            pl.BlockSpec((_BM, D), lambda i: (i, 0)),
            pl.BlockSpec((_BM, 2 * D), lambda i: (i, 0)),
            pl.BlockSpec((_BM, D), lambda i: (i, 0)),
            pl.BlockSpec((1, H), lambda i: (0, 0)),
            pl.BlockSpec((1, H), lambda i: (0, 0)),
        ],
        out_shape=[
            jax.ShapeDtypeStruct((N, D), jnp.float32),
            jax.ShapeDtypeStruct((N, 2 * D), jnp.float32),
            jax.ShapeDtypeStruct((N, D), jnp.float32),
            jax.ShapeDtypeStruct((1, H), jnp.float32),
            jax.ShapeDtypeStruct((1, H), jnp.float32),
        ],
    )(h, Wq, bq.reshape(1, D), Wkv, bkv.reshape(1, 2 * D),
      Ws, bs.reshape(1, D), selo)


def _weight_body(qd_ref, ks_ref, vs_ref, mq_ref, mk_ref, sel_ref, exp_ref,
                 ex_ref, w_ref):
    prod = qd_ref[...] * ks_ref[...]
    lg = jnp.dot(prod, sel_ref[...], preferred_element_type=jnp.float32)
    shift = jnp.maximum(
        0.25 * jnp.sqrt(mq_ref[...] * mk_ref[...]) - 60.0, 0.0
    )
    ex = jnp.exp(lg - shift)
    ex_ref[:, :H] = ex
    ex_ref[:, H:] = jnp.zeros_like(ex)
    w_ref[...] = vs_ref[...] * jnp.dot(
        ex, exp_ref[...], preferred_element_type=jnp.float32
    )


def _weight(qd, kvs, mq2, mk2, sel, expand):
    return pl.pallas_call(
        _weight_body,
        grid=(E2 // _BE,),
        in_specs=[
            pl.BlockSpec((_BE, D), lambda i: (i, 0)),
            pl.BlockSpec((_BE, D), lambda i: (i, 0)),
            pl.BlockSpec((_BE, D), lambda i: (i, 1)),
            pl.BlockSpec((1, H), lambda i: (0, 0)),
            pl.BlockSpec((1, H), lambda i: (0, 0)),
            pl.BlockSpec((D, H), lambda i: (0, 0)),
            pl.BlockSpec((H, D), lambda i: (0, 0)),
        ],
        out_specs=[
            pl.BlockSpec((_BE, 16), lambda i: (i, 0)),
            pl.BlockSpec((_BE, D), lambda i: (i, 0)),
        ],
        out_shape=[
            jax.ShapeDtypeStruct((E2, 16), jnp.float32),
            jax.ShapeDtypeStruct((E2, D), jnp.float32),
        ],
    )(qd, kvs, kvs, mq2, mk2, sel, expand)


def _finish_body(pa0_ref, pa1_ref, pb0_ref, pb1_ref,
                 da0_ref, da1_ref, db0_ref, db1_ref,
                 s_ref, exp_ref, o_ref):
    den = (da0_ref[...] + da1_ref[...] + db0_ref[...] + db1_ref[...])
    den = den[:, :H] + 1e-16
    r = jnp.dot(1.0 / den, exp_ref[...], preferred_element_type=jnp.float32)
    p = pa0_ref[...] + pa1_ref[...] + pb0_ref[...] + pb1_ref[...]
    o_ref[...] = jax.nn.relu(p * r + s_ref[...])


def _finish(pa0, pa1, pb0, pb1, da0, da1, db0, db1, s, expand):
    bd = pl.BlockSpec((_BM, D), lambda i: (i, 0))
    b16 = pl.BlockSpec((_BM, 16), lambda i: (i, 0))
    return pl.pallas_call(
        _finish_body,
        grid=(N // _BM,),
        in_specs=[bd, bd, bd, bd, b16, b16, b16, b16, bd,
                  pl.BlockSpec((H, D), lambda i: (0, 0))],
        out_specs=bd,
        out_shape=jax.ShapeDtypeStruct((N, D), jnp.float32),
    )(pa0, pa1, pb0, pb1, da0, da1, db0, db1, s, expand)


def kernel(x, edge_index, params):
    dst = edge_index[1]
    src = edge_index[0]
    dst3 = [dst[h * E2:(h + 1) * E2].reshape(NW, NJ, CH) for h in range(2)]
    src3 = [src[h * E2:(h + 1) * E2].reshape(NW, NJ, CH) for h in range(2)]
    z16 = jnp.zeros((NPS, 16), jnp.float32)
    z128 = jnp.zeros((NPS, D), jnp.float32)
    heads = np.arange(D) // C
    mask = (heads[:, None] == np.arange(H)[None, :])
    sel = jnp.asarray(mask / np.sqrt(C), jnp.float32)
    selo = jnp.asarray(mask.astype(np.float32))
    expand = jnp.asarray(mask.T.astype(np.float32))

    h = _linear(x, params['W_in'], params['b_in'])
    for l in range(L):
        Wkv = jnp.concatenate(
            [params['Wk%d' % l], params['Wv%d' % l]], axis=1
        )
        bkv = jnp.concatenate([params['bk%d' % l], params['bv%d' % l]])
        q, kv, s, mq2, mk2 = _proj3(
            h, params['Wq%d' % l], params['bq%d' % l], Wkv, bkv,
            params['Ws%d' % l], params['bs%d' % l], selo)
        qdA, kvsA = _sc_gather(q, kv, dst3[0], src3[0])
        qdB, kvsB = _sc_gather(q, kv, dst3[1], src3[1])
        exA, wA = _weight(qdA, kvsA, mq2, mk2, sel, expand)
        denA, outA = _sc_scatter(exA, wA, dst3[0], z16, z128)
        exB, wB = _weight(qdB, kvsB, mq2, mk2, sel, expand)
        denB, outB = _sc_scatter(exB, wB, dst3[1], z16, z128)
        h = _finish(outA[0, :N], outA[1, :N], outB[0, :N], outB[1, :N],
                    denA[0, :N], denA[1, :N], denB[0, :N], denB[1, :N],
                    s, expand)
    return _linear(h, params['W_out'], params['b_out'])
